# baseline (speedup 1.0000x reference)
import functools

import jax
import jax.numpy as jnp
from jax import lax
from jax.experimental import pallas as pl
from jax.experimental.pallas import tpu as pltpu
from jax.experimental.pallas import tpu_sc as plsc

N_NODES = 2048
N_EDGES = 65536
NUM_CORES = 2
NUM_SUBCORES = 16
NW = NUM_CORES * NUM_SUBCORES
EDGES_PER_TILE = N_EDGES // NW
CHUNK = 128
NCHUNK = EDGES_PER_TILE // CHUNK
ROWS_PER_SUB = N_NODES // NUM_SUBCORES
BLK = EDGES_PER_TILE


def _perm_idx(idx, width, blk):
    k = 128 // width
    return (idx.reshape(N_EDGES // blk, k, blk // k)
               .swapaxes(1, 2)
               .reshape(NW, NCHUNK, CHUNK))


def _sc_scatter_add(msg, idx3, zeros, width, nstage):
    mesh = plsc.VectorSubcoreMesh(core_axis_name="c", subcore_axis_name="s")
    stage_rows = EDGES_PER_TILE // nstage
    stage_chunks = NCHUNK // nstage

    @functools.partial(
        pl.kernel,
        mesh=mesh,
        out_type=jax.ShapeDtypeStruct((NUM_CORES, N_NODES, width), jnp.float32),
        compiler_params=pltpu.CompilerParams(use_tc_tiling_on_sc=False),
        scratch_types=[
            pltpu.VMEM((NCHUNK, CHUNK), jnp.int32),
            pltpu.VMEM((stage_rows, width), jnp.float32),
            pltpu.VMEM((ROWS_PER_SUB, width), jnp.float32),
            pltpu.VMEM_SHARED((N_NODES, width), jnp.float32),
            pltpu.SemaphoreType.DMA,
        ],
    )
    def k(msg_hbm, idx_hbm, zero_hbm, out_hbm, idx_v, msg_v, buf_v, acc_sh, sem):
        c = lax.axis_index("c")
        s = lax.axis_index("s")
        wid = c * NUM_SUBCORES + s
        base = wid * EDGES_PER_TILE
        r0 = s * ROWS_PER_SUB
        pltpu.sync_copy(zero_hbm.at[pl.ds(r0, ROWS_PER_SUB)], buf_v)
        pltpu.sync_copy(buf_v, acc_sh.at[pl.ds(r0, ROWS_PER_SUB)])
        pltpu.sync_copy(idx_hbm.at[wid], idx_v)
        plsc.subcore_barrier()
        for st in range(nstage):
            pltpu.sync_copy(
                msg_hbm.at[pl.ds(base + st * stage_rows, stage_rows)], msg_v)
            cps = [
                pltpu.async_copy(
                    msg_v.at[pl.ds(j * CHUNK, CHUNK)],
                    acc_sh.at[idx_v.at[st * stage_chunks + j]],
                    sem,
                    add=True,
                )
                for j in range(stage_chunks)
            ]
            for cp in cps:
                cp.wait()
        plsc.subcore_barrier()
        pltpu.sync_copy(acc_sh.at[pl.ds(r0, ROWS_PER_SUB)], buf_v)
        pltpu.sync_copy(buf_v, out_hbm.at[c, pl.ds(r0, ROWS_PER_SUB)])

    return k(msg, idx3, zeros)


def _sc_gather(table, idx3):
    width = table.shape[1]
    mesh = plsc.VectorSubcoreMesh(core_axis_name="c", subcore_axis_name="s")

    @functools.partial(
        pl.kernel,
        mesh=mesh,
        out_type=jax.ShapeDtypeStruct((N_EDGES, width), jnp.float32),
        compiler_params=pltpu.CompilerParams(use_tc_tiling_on_sc=False),
        scratch_types=[
            pltpu.VMEM((NCHUNK, CHUNK), jnp.int32),
            pltpu.VMEM((EDGES_PER_TILE, width), jnp.float32),
            pltpu.SemaphoreType.DMA,
        ],
    )
    def k(tab_hbm, idx_hbm, out_hbm, idx_v, rows_v, sem):
        c = lax.axis_index("c")
        s = lax.axis_index("s")
        wid = c * NUM_SUBCORES + s
        base = wid * EDGES_PER_TILE
        pltpu.sync_copy(idx_hbm.at[wid], idx_v)
        copies = [
            pltpu.async_copy(
                tab_hbm.at[idx_v.at[j]],
                rows_v.at[pl.ds(j * CHUNK, CHUNK)],
                sem,
            )
            for j in range(NCHUNK)
        ]
        for cp in copies:
            cp.wait()
        pltpu.sync_copy(rows_v, out_hbm.at[pl.ds(base, EDGES_PER_TILE)])

    return k(table, idx3)


def _pack_lanes(x, width):
    k = 128 // width
    rows = x.shape[0] // k
    return jnp.concatenate([x[j * rows:(j + 1) * rows, :] for j in range(k)],
                           axis=1)


def _unpack_lanes(xp, width):
    k = 128 // width
    return jnp.concatenate([xp[:, j * width:(j + 1) * width] for j in range(k)],
                           axis=0)


MBLK1 = 8192
MBLK = 4096


def _tc_msg1(ea, lin1_W, lin1_b):
    def body(ea_ref, w_ref, b_ref, out_ref):
        m = lax.dot_general(ea_ref[...].astype(jnp.bfloat16),
                            w_ref[...].astype(jnp.bfloat16),
                            (((0,), (0,)), ((), ())),
                            preferred_element_type=jnp.float32)
        m = jnp.maximum(m + b_ref[...], 0.0)
        col = lax.broadcasted_iota(jnp.int32, (MBLK1, 32), 1)
        cnt = jnp.where(col == 0, 1.0, 0.0)
        aug = jnp.concatenate([m, cnt], axis=1)
        out_ref[...] = _pack_lanes(aug, 64)

    return pl.pallas_call(
        body,
        grid=(N_EDGES // MBLK1,),
        in_specs=[
            pl.BlockSpec((4, MBLK1), lambda i: (0, i)),
            pl.BlockSpec((4, 32), lambda i: (0, 0)),
            pl.BlockSpec((1, 32), lambda i: (0, 0)),
        ],
        out_specs=pl.BlockSpec((MBLK1 * 64 // 128, 128), lambda i: (i, 0)),
        out_shape=jax.ShapeDtypeStruct((N_EDGES * 64 // 128, 128), jnp.float32),
    )(ea, lin1_W, lin1_b)


def _tc_h1(partials, root1_W, root1_b):
    def body(p_ref, rw_ref, rb_ref, h_ref, inv_ref):
        p = p_ref[0] + p_ref[1]
        deg = p[:, 32:33]
        inv = 1.0 / jnp.maximum(deg, 1.0)
        h = p[:, 0:32] * inv + rw_ref[...] + rb_ref[...]
        h_ref[...] = jnp.maximum(h, 0.0)
        inv_ref[...] = inv

    return pl.pallas_call(
        body,
        out_shape=[
            jax.ShapeDtypeStruct((N_NODES, 32), jnp.float32),
            jax.ShapeDtypeStruct((N_NODES, 1), jnp.float32),
        ],
    )(partials, root1_W, root1_b)


def _tc_msg(ea, xj_p, A, B, WWcat, R, c_in, c_all):
    zw = 4 * c_in

    def body(ea_ref, xj_ref, a_ref, b_ref, ww_ref, r_ref, out_ref):
        ea16 = ea_ref[...].astype(jnp.bfloat16)
        za = lax.dot_general(ea16, a_ref[...].astype(jnp.bfloat16),
                             (((0,), (0,)), ((), ())),
                             preferred_element_type=jnp.float32)
        xj16 = _unpack_lanes(xj_ref[...], c_in).astype(jnp.bfloat16)
        zb = jnp.dot(xj16, b_ref[...].astype(jnp.bfloat16),
                     preferred_element_type=jnp.float32)
        z = (za.astype(jnp.bfloat16)) * (zb.astype(jnp.bfloat16))
        feat = jnp.concatenate([z, xj16], axis=1)
        p = jnp.dot(feat, ww_ref[...].astype(jnp.bfloat16),
                    preferred_element_type=jnp.float32)
        msg = jnp.dot(jnp.maximum(p, 0.0).astype(jnp.bfloat16),
                      r_ref[...].astype(jnp.bfloat16),
                      preferred_element_type=jnp.float32)
        out_ref[...] = _pack_lanes(msg, 16)

    return pl.pallas_call(
        body,
        grid=(N_EDGES // MBLK,),
        in_specs=[
            pl.BlockSpec((4, MBLK), lambda i: (0, i)),
            pl.BlockSpec((MBLK * c_in // 128, 128), lambda i: (i, 0)),
            pl.BlockSpec((4, zw), lambda i: (0, 0)),
            pl.BlockSpec((c_in, zw), lambda i: (0, 0)),
            pl.BlockSpec((zw + c_in, c_all), lambda i: (0, 0)),
            pl.BlockSpec((c_all, 16), lambda i: (0, 0)),
        ],
        out_specs=pl.BlockSpec((MBLK * 16 // 128, 128), lambda i: (i, 0)),
        out_shape=jax.ShapeDtypeStruct((N_EDGES * 16 // 128, 128), jnp.float32),
    )(ea, xj_p, A, B, WWcat, R)


def _tc_update(partials, h_prev, invdeg, root_W, root_b, c_out):
    def body(p_ref, h_ref, inv_ref, rw_ref, rb_ref, out_ref):
        p = p_ref[0] + p_ref[1]
        mean = p[:, 0:c_out] * inv_ref[...]
        h = jnp.dot(h_ref[...], rw_ref[...], preferred_element_type=jnp.float32,
                    precision=lax.Precision.HIGHEST)
        out_ref[...] = jnp.maximum(h + mean + rb_ref[...], 0.0)

    return pl.pallas_call(
        body,
        out_shape=jax.ShapeDtypeStruct((N_NODES, c_out), jnp.float32),
    )(partials, h_prev, invdeg, root_W, root_b)


def _tc_cbt(h3, h3T):
    RBLK = 256

    def body(a_ref, bt_ref, out_ref):
        acc = jnp.zeros((RBLK, N_NODES), jnp.float32)
        for d in range(8):
            acc = acc + jnp.abs(a_ref[:, d : d + 1] - bt_ref[d : d + 1, :])
        out_ref[...] = acc

    return pl.pallas_call(
        body,
        grid=(N_NODES // RBLK,),
        in_specs=[
            pl.BlockSpec((RBLK, 8), lambda i: (i, 0)),
            pl.BlockSpec((8, N_NODES), lambda i: (0, 0)),
        ],
        out_specs=pl.BlockSpec((RBLK, N_NODES), lambda i: (i, 0)),
        out_shape=jax.ShapeDtypeStruct((N_NODES, N_NODES), jnp.float32),
    )(h3, h3T)


def kernel(x, edge_attr, edge_index, lin1_W, lin1_b, root1_W, root1_b,
           lin2_W, lin2_b, root2_W, root2_b, lin3_W, lin3_b, root3_W, root3_b):
    del x
    src = edge_index[0]
    dst = edge_index[1]
    dst_w16 = _perm_idx(dst, 16, 4096)
    src_w32 = _perm_idx(src, 32, 4096)
    src_w16 = _perm_idx(src, 16, 4096)

    zeros64 = jnp.zeros((N_NODES, 64), jnp.float32)
    zeros16 = jnp.zeros((N_NODES, 16), jnp.float32)

    def bilinear_consts(lin_W, lin_b, ci, co):
        A = jnp.kron(jnp.eye(4, dtype=jnp.float32), jnp.ones((1, ci), jnp.float32))
        B = jnp.kron(jnp.ones((1, 4), jnp.float32), jnp.eye(ci, dtype=jnp.float32))
        W3 = lin_W.reshape(4, ci, co)
        eye = jnp.eye(ci, dtype=jnp.float32)
        WW = (W3[:, None, :, :] * eye[None, :, :, None]).reshape(4 * ci, ci * co)
        BB = (eye[:, :, None] * lin_b.reshape(ci, co)[None]).reshape(ci, ci * co)
        R = jnp.kron(jnp.ones((ci, 1), jnp.float32), jnp.eye(co, dtype=jnp.float32))
        return A, B, jnp.concatenate([WW, BB], axis=0), R

    A2, B2, WWcat2, R2 = bilinear_consts(lin2_W, lin2_b, 32, 16)
    A3, B3, WWcat3, R3 = bilinear_consts(lin3_W, lin3_b, 16, 8)
    R3 = jnp.pad(R3, ((0, 0), (0, 8)))

    halves = dst.reshape(N_EDGES // MBLK1, 2, MBLK1 // 2).astype(jnp.float32)
    inter = lax.complex(halves[:, 0], halves[:, 1]).view(jnp.float32)
    dst_w64 = inter.reshape(NW, NCHUNK, CHUNK).astype(jnp.int32)

    eaT = edge_attr.T
    msg1 = _tc_msg1(eaT, lin1_W, lin1_b.reshape(1, 32))
    part1 = _sc_scatter_add(msg1.reshape(N_EDGES, 64), dst_w64, zeros64, 64, 2)
    h1, invdeg = _tc_h1(part1, root1_W, root1_b.reshape(1, 32))

    xj2 = _sc_gather(h1, src_w32).reshape(N_EDGES * 32 // 128, 128)
    msg2 = _tc_msg(eaT, xj2, A2, B2, WWcat2, R2, 32, 512)
    part2 = _sc_scatter_add(msg2.reshape(N_EDGES, 16), dst_w16, zeros16, 16, 1)
    h2 = _tc_update(part2, h1, invdeg, root2_W, root2_b.reshape(1, 16), 16)

    xj3 = _sc_gather(h2, src_w16).reshape(N_EDGES * 16 // 128, 128)
    msg3 = _tc_msg(eaT, xj3, A3, B3, WWcat3, R3, 16, 128)
    part3 = _sc_scatter_add(msg3.reshape(N_EDGES, 16), dst_w16, zeros16, 16, 1)
    h3 = _tc_update(part3, h2, invdeg, root3_W, root3_b.reshape(1, 8), 8)

    return _tc_cbt(h3, h3.T)


# speedup vs baseline: 1.0407x; 1.0407x over previous
import functools

import jax
import jax.numpy as jnp
from jax import lax
from jax.experimental import pallas as pl
from jax.experimental.pallas import tpu as pltpu
from jax.experimental.pallas import tpu_sc as plsc

N_NODES = 2048
N_EDGES = 65536
NUM_CORES = 2
NUM_SUBCORES = 16
NW = NUM_CORES * NUM_SUBCORES
EDGES_PER_TILE = N_EDGES // NW
CHUNK = 128
NCHUNK = EDGES_PER_TILE // CHUNK
ROWS_PER_SUB = N_NODES // NUM_SUBCORES
BLK = EDGES_PER_TILE


def _perm_idx(idx, width, blk):
    k = 128 // width
    return (idx.reshape(N_EDGES // blk, k, blk // k)
               .swapaxes(1, 2)
               .reshape(NW, NCHUNK, CHUNK))


def _sc_scatter_add(msg, idx3, zeros, width, nstage):
    mesh = plsc.VectorSubcoreMesh(core_axis_name="c", subcore_axis_name="s")
    stage_rows = EDGES_PER_TILE // nstage
    stage_chunks = NCHUNK // nstage

    @functools.partial(
        pl.kernel,
        mesh=mesh,
        out_type=jax.ShapeDtypeStruct((NUM_CORES, N_NODES, width), jnp.float32),
        compiler_params=pltpu.CompilerParams(use_tc_tiling_on_sc=False),
        scratch_types=[
            pltpu.VMEM((NCHUNK, CHUNK), jnp.int32),
            pltpu.VMEM((stage_rows, width), jnp.float32),
            pltpu.VMEM((ROWS_PER_SUB, width), jnp.float32),
            pltpu.VMEM_SHARED((N_NODES, width), jnp.float32),
            pltpu.SemaphoreType.DMA,
        ],
    )
    def k(msg_hbm, idx_hbm, zero_hbm, out_hbm, idx_v, msg_v, buf_v, acc_sh, sem):
        c = lax.axis_index("c")
        s = lax.axis_index("s")
        wid = c * NUM_SUBCORES + s
        base = wid * EDGES_PER_TILE
        r0 = s * ROWS_PER_SUB
        pltpu.sync_copy(zero_hbm.at[pl.ds(r0, ROWS_PER_SUB)], buf_v)
        pltpu.sync_copy(buf_v, acc_sh.at[pl.ds(r0, ROWS_PER_SUB)])
        pltpu.sync_copy(idx_hbm.at[wid], idx_v)
        plsc.subcore_barrier()
        for st in range(nstage):
            pltpu.sync_copy(
                msg_hbm.at[pl.ds(base + st * stage_rows, stage_rows)], msg_v)
            cps = [
                pltpu.async_copy(
                    msg_v.at[pl.ds(j * CHUNK, CHUNK)],
                    acc_sh.at[idx_v.at[st * stage_chunks + j]],
                    sem,
                    add=True,
                )
                for j in range(stage_chunks)
            ]
            for cp in cps:
                cp.wait()
        plsc.subcore_barrier()
        pltpu.sync_copy(acc_sh.at[pl.ds(r0, ROWS_PER_SUB)], buf_v)
        pltpu.sync_copy(buf_v, out_hbm.at[c, pl.ds(r0, ROWS_PER_SUB)])

    return k(msg, idx3, zeros)


def _sc_gather(table, idx3):
    width = table.shape[1]
    mesh = plsc.VectorSubcoreMesh(core_axis_name="c", subcore_axis_name="s")

    @functools.partial(
        pl.kernel,
        mesh=mesh,
        out_type=jax.ShapeDtypeStruct((N_EDGES, width), jnp.float32),
        compiler_params=pltpu.CompilerParams(use_tc_tiling_on_sc=False),
        scratch_types=[
            pltpu.VMEM((NCHUNK, CHUNK), jnp.int32),
            pltpu.VMEM((EDGES_PER_TILE, width), jnp.float32),
            pltpu.SemaphoreType.DMA,
        ],
    )
    def k(tab_hbm, idx_hbm, out_hbm, idx_v, rows_v, sem):
        c = lax.axis_index("c")
        s = lax.axis_index("s")
        wid = c * NUM_SUBCORES + s
        base = wid * EDGES_PER_TILE
        pltpu.sync_copy(idx_hbm.at[wid], idx_v)
        copies = [
            pltpu.async_copy(
                tab_hbm.at[idx_v.at[j]],
                rows_v.at[pl.ds(j * CHUNK, CHUNK)],
                sem,
            )
            for j in range(NCHUNK)
        ]
        for cp in copies:
            cp.wait()
        pltpu.sync_copy(rows_v, out_hbm.at[pl.ds(base, EDGES_PER_TILE)])

    return k(table, idx3)


def _pack_lanes(x, width):
    k = 128 // width
    rows = x.shape[0] // k
    return jnp.concatenate([x[j * rows:(j + 1) * rows, :] for j in range(k)],
                           axis=1)


def _unpack_lanes(xp, width):
    k = 128 // width
    return jnp.concatenate([xp[:, j * width:(j + 1) * width] for j in range(k)],
                           axis=0)


MBLK1 = 8192
MBLK = 4096


def _tc_msg1(ea, lin1_W, lin1_b):
    def body(ea_ref, w_ref, b_ref, out_ref):
        m = lax.dot_general(ea_ref[...].astype(jnp.bfloat16),
                            w_ref[...].astype(jnp.bfloat16),
                            (((0,), (0,)), ((), ())),
                            preferred_element_type=jnp.float32)
        m = jnp.maximum(m + b_ref[...], 0.0)
        col = lax.broadcasted_iota(jnp.int32, (MBLK1, 32), 1)
        cnt = jnp.where(col == 0, 1.0, 0.0)
        aug = jnp.concatenate([m, cnt], axis=1)
        out_ref[...] = _pack_lanes(aug, 64)

    return pl.pallas_call(
        body,
        grid=(N_EDGES // MBLK1,),
        in_specs=[
            pl.BlockSpec((4, MBLK1), lambda i: (0, i)),
            pl.BlockSpec((4, 32), lambda i: (0, 0)),
            pl.BlockSpec((1, 32), lambda i: (0, 0)),
        ],
        out_specs=pl.BlockSpec((MBLK1 * 64 // 128, 128), lambda i: (i, 0)),
        out_shape=jax.ShapeDtypeStruct((N_EDGES * 64 // 128, 128), jnp.float32),
    )(ea, lin1_W, lin1_b)


def _tc_h1(partials, root1_W, root1_b):
    def body(p_ref, rw_ref, rb_ref, h_ref, inv_ref):
        p = p_ref[0] + p_ref[1]
        deg = p[:, 32:33]
        inv = 1.0 / jnp.maximum(deg, 1.0)
        h = p[:, 0:32] * inv + rw_ref[...] + rb_ref[...]
        h_ref[...] = jnp.maximum(h, 0.0)
        inv_ref[...] = inv

    return pl.pallas_call(
        body,
        out_shape=[
            jax.ShapeDtypeStruct((N_NODES, 32), jnp.float32),
            jax.ShapeDtypeStruct((N_NODES, 1), jnp.float32),
        ],
    )(partials, root1_W, root1_b)


def _tc_msg(ea, xj_p, A, B, WWcat, R, c_in, c_all):
    zw = 4 * c_in

    def body(ea_ref, xj_ref, a_ref, b_ref, ww_ref, r_ref, out_ref):
        ea16 = ea_ref[...].astype(jnp.bfloat16)
        za = lax.dot_general(ea16, a_ref[...].astype(jnp.bfloat16),
                             (((0,), (0,)), ((), ())),
                             preferred_element_type=jnp.float32)
        xj16 = _unpack_lanes(xj_ref[...], c_in).astype(jnp.bfloat16)
        zb = jnp.dot(xj16, b_ref[...].astype(jnp.bfloat16),
                     preferred_element_type=jnp.float32)
        z = (za.astype(jnp.bfloat16)) * (zb.astype(jnp.bfloat16))
        feat = jnp.concatenate([z, xj16], axis=1)
        p = jnp.dot(feat, ww_ref[...].astype(jnp.bfloat16),
                    preferred_element_type=jnp.float32)
        msg = jnp.dot(jnp.maximum(p, 0.0).astype(jnp.bfloat16),
                      r_ref[...].astype(jnp.bfloat16),
                      preferred_element_type=jnp.float32)
        out_ref[...] = _pack_lanes(msg, 16)

    return pl.pallas_call(
        body,
        grid=(N_EDGES // MBLK,),
        in_specs=[
            pl.BlockSpec((4, MBLK), lambda i: (0, i)),
            pl.BlockSpec((MBLK * c_in // 128, 128), lambda i: (i, 0)),
            pl.BlockSpec((4, zw), lambda i: (0, 0)),
            pl.BlockSpec((c_in, zw), lambda i: (0, 0)),
            pl.BlockSpec((zw + c_in, c_all), lambda i: (0, 0)),
            pl.BlockSpec((c_all, 16), lambda i: (0, 0)),
        ],
        out_specs=pl.BlockSpec((MBLK * 16 // 128, 128), lambda i: (i, 0)),
        out_shape=jax.ShapeDtypeStruct((N_EDGES * 16 // 128, 128), jnp.float32),
    )(ea, xj_p, A, B, WWcat, R)


def _tc_update(partials, h_prev, invdeg, root_W, root_b, c_out):
    def body(p_ref, h_ref, inv_ref, rw_ref, rb_ref, out_ref):
        p = p_ref[0] + p_ref[1]
        mean = p[:, 0:c_out] * inv_ref[...]
        h = jnp.dot(h_ref[...], rw_ref[...], preferred_element_type=jnp.float32,
                    precision=lax.Precision.HIGHEST)
        out_ref[...] = jnp.maximum(h + mean + rb_ref[...], 0.0)

    return pl.pallas_call(
        body,
        out_shape=jax.ShapeDtypeStruct((N_NODES, c_out), jnp.float32),
    )(partials, h_prev, invdeg, root_W, root_b)


def _tc_cbt(h3, h3T):
    RBLK = 256

    def body(a_ref, bt_ref, out_ref):
        acc = jnp.zeros((RBLK, N_NODES), jnp.float32)
        for d in range(8):
            acc = acc + jnp.abs(a_ref[:, d : d + 1] - bt_ref[d : d + 1, :])
        out_ref[...] = acc

    return pl.pallas_call(
        body,
        grid=(N_NODES // RBLK,),
        in_specs=[
            pl.BlockSpec((RBLK, 8), lambda i: (i, 0)),
            pl.BlockSpec((8, N_NODES), lambda i: (0, 0)),
        ],
        out_specs=pl.BlockSpec((RBLK, N_NODES), lambda i: (i, 0)),
        out_shape=jax.ShapeDtypeStruct((N_NODES, N_NODES), jnp.float32),
    )(h3, h3T)


def kernel(x, edge_attr, edge_index, lin1_W, lin1_b, root1_W, root1_b,
           lin2_W, lin2_b, root2_W, root2_b, lin3_W, lin3_b, root3_W, root3_b):
    del x
    src = edge_index[0]
    dst = edge_index[1]
    dst_w64 = _perm_idx(dst, 64, 8192)
    dst_w16 = _perm_idx(dst, 16, 4096)
    src_w32 = _perm_idx(src, 32, 4096)
    src_w16 = _perm_idx(src, 16, 4096)

    zeros64 = jnp.zeros((N_NODES, 64), jnp.float32)
    zeros16 = jnp.zeros((N_NODES, 16), jnp.float32)

    def bilinear_consts(lin_W, lin_b, ci, co):
        A = jnp.kron(jnp.eye(4, dtype=jnp.float32), jnp.ones((1, ci), jnp.float32))
        B = jnp.kron(jnp.ones((1, 4), jnp.float32), jnp.eye(ci, dtype=jnp.float32))
        W3 = lin_W.reshape(4, ci, co)
        eye = jnp.eye(ci, dtype=jnp.float32)
        WW = (W3[:, None, :, :] * eye[None, :, :, None]).reshape(4 * ci, ci * co)
        BB = (eye[:, :, None] * lin_b.reshape(ci, co)[None]).reshape(ci, ci * co)
        R = jnp.kron(jnp.ones((ci, 1), jnp.float32), jnp.eye(co, dtype=jnp.float32))
        return A, B, jnp.concatenate([WW, BB], axis=0), R

    A2, B2, WWcat2, R2 = bilinear_consts(lin2_W, lin2_b, 32, 16)
    A3, B3, WWcat3, R3 = bilinear_consts(lin3_W, lin3_b, 16, 8)
    R3 = jnp.pad(R3, ((0, 0), (0, 8)))

    eaT = edge_attr.T
    msg1 = _tc_msg1(eaT, lin1_W, lin1_b.reshape(1, 32))
    part1 = _sc_scatter_add(msg1.reshape(N_EDGES, 64), dst_w64, zeros64, 64, 2)
    h1, invdeg = _tc_h1(part1, root1_W, root1_b.reshape(1, 32))

    xj2 = _sc_gather(h1, src_w32).reshape(N_EDGES * 32 // 128, 128)
    msg2 = _tc_msg(eaT, xj2, A2, B2, WWcat2, R2, 32, 512)
    part2 = _sc_scatter_add(msg2.reshape(N_EDGES, 16), dst_w16, zeros16, 16, 1)
    h2 = _tc_update(part2, h1, invdeg, root2_W, root2_b.reshape(1, 16), 16)

    xj3 = _sc_gather(h2, src_w16).reshape(N_EDGES * 16 // 128, 128)
    msg3 = _tc_msg(eaT, xj3, A3, B3, WWcat3, R3, 16, 128)
    part3 = _sc_scatter_add(msg3.reshape(N_EDGES, 16), dst_w16, zeros16, 16, 1)
    h3 = _tc_update(part3, h2, invdeg, root3_W, root3_b.reshape(1, 8), 8)

    return _tc_cbt(h3, h3.T)
